# baseline (speedup 1.0000x reference)
import math

import jax
import jax.numpy as jnp
import numpy as np
from jax.experimental import pallas as pl
from jax.experimental.pallas import tpu as pltpu

N = 8192
NMOL = 256
HIDDEN = 128
NF = 128
NI = 6
NG = 50
NGP = 128
CUTOFF = 10.0
NT = 12
TR = 32
TC = 32
NRB = N // TR
RB = 128
NZ = 100
LOG2 = math.log(2.0)
HI = jax.lax.Precision.HIGHEST
_OFFS_NP = np.linspace(0.0, CUTOFF, NG).astype(np.float32)
_DELTA = _OFFS_NP[1] - _OFFS_NP[0]
_COEFF = float(np.float32(-0.5) / (_DELTA * _DELTA))


def _ssp(x):
    return jnp.maximum(x, 0.0) + jnp.log1p(jnp.exp(-jnp.abs(x))) - LOG2


def _embed_body(z_ref, emb_ref, o_ref):
    z = z_ref[...]
    oh = (z == jax.lax.broadcasted_iota(jnp.int32, (RB, NZ), 1)).astype(jnp.float32)
    o_ref[...] = jax.lax.dot(oh, emb_ref[...], precision=HI)


def _embed(z, emb):
    return pl.pallas_call(
        _embed_body,
        grid=(N // RB,),
        in_specs=[
            pl.BlockSpec((RB, 1), lambda i: (i, 0)),
            pl.BlockSpec((NZ, HIDDEN), lambda i: (0, 0)),
        ],
        out_specs=pl.BlockSpec((RB, HIDDEN), lambda i: (i, 0)),
        out_shape=jax.ShapeDtypeStruct((N, HIDDEN), jnp.float32),
    )(z.reshape(N, 1), emb)


def _mm_body(x_ref, w_ref, o_ref):
    o_ref[...] = jax.lax.dot(x_ref[...], w_ref[...], precision=HI)


def _xs(h, w):
    return pl.pallas_call(
        _mm_body,
        grid=(N // RB,),
        in_specs=[
            pl.BlockSpec((RB, HIDDEN), lambda i: (i, 0)),
            pl.BlockSpec((HIDDEN, NF), lambda i: (0, 0)),
        ],
        out_specs=pl.BlockSpec((RB, NF), lambda i: (i, 0)),
        out_shape=jax.ShapeDtypeStruct((N, NF), jnp.float32),
    )(h, w)


def _pair_body(cs_ref, cn_ref, a_ref, b_ref, bat_ref, xs_ref,
               w1_ref, b1_ref, w2_ref, b2_ref, offs_ref, o_ref):
    i = pl.program_id(0)
    r0 = i * TR
    a_blk = a_ref[...]
    bcol = bat_ref[pl.ds(r0, TR), :]
    row_ids = r0 + jax.lax.broadcasted_iota(jnp.int32, (TR, TC), 0)
    offs = offs_ref[...]
    coeff = _COEFF
    w1 = w1_ref[...]
    b1 = b1_ref[...]
    w2 = w2_ref[...]
    b2 = b2_ref[...]

    def body(t, acc):
        c0 = t * TC
        b_j = b_ref[pl.ds(c0, TC), :]
        d2 = jax.lax.dot_general(
            a_blk, b_j, (((1,), (1,)), ((), ())), precision=HI)
        brow = bat_ref[pl.ds(c0, TC), :].reshape(1, TC)
        col_ids = c0 + jax.lax.broadcasted_iota(jnp.int32, (TR, TC), 1)
        mask = (bcol == brow) & (d2 <= CUTOFF * CUTOFF) & (row_ids != col_ids)
        dm = jnp.where(mask, d2, 1e9)
        w = jnp.sqrt(jnp.maximum(dm, 0.0))
        cc = jnp.where(dm < 1e8,
                       0.5 * (jnp.cos(w * (math.pi / CUTOFF)) + 1.0), 0.0)
        w3 = w.reshape(TR, TC, 1)
        ea = jnp.exp(coeff * (w3 - offs.reshape(1, 1, NGP)) ** 2)
        ea = ea.astype(jnp.bfloat16).reshape(TR * TC, NGP)
        a1 = jax.lax.dot(ea, w1, preferred_element_type=jnp.float32) + b1
        wf = jax.lax.dot(_ssp(a1).astype(jnp.bfloat16), w2,
                         preferred_element_type=jnp.float32) + b2
        wf3 = wf.reshape(TR, TC, NF) * cc.reshape(TR, TC, 1)
        xsj = xs_ref[pl.ds(c0, TC), :]
        msg = wf3 * xsj[None, :, :]
        return acc + jnp.sum(msg, axis=1)

    t0 = cs_ref[i]
    acc = jax.lax.fori_loop(t0, t0 + cn_ref[i], body,
                            jnp.zeros((TR, NF), jnp.float32))
    o_ref[...] = acc


def _pair(cstart, cnum, A, B, batf, xs, w1, b1, w2, b2, offs):
    grid_spec = pltpu.PrefetchScalarGridSpec(
        num_scalar_prefetch=2,
        grid=(NRB,),
        in_specs=[
            pl.BlockSpec((TR, 8), lambda i, cs, cn: (i, 0)),
            pl.BlockSpec((N, 8), lambda i, cs, cn: (0, 0)),
            pl.BlockSpec((N, 1), lambda i, cs, cn: (0, 0)),
            pl.BlockSpec((N, NF), lambda i, cs, cn: (0, 0)),
            pl.BlockSpec((NGP, NF), lambda i, cs, cn: (0, 0)),
            pl.BlockSpec((1, NF), lambda i, cs, cn: (0, 0)),
            pl.BlockSpec((NF, NF), lambda i, cs, cn: (0, 0)),
            pl.BlockSpec((1, NF), lambda i, cs, cn: (0, 0)),
            pl.BlockSpec((1, NGP), lambda i, cs, cn: (0, 0)),
        ],
        out_specs=pl.BlockSpec((TR, NF), lambda i, cs, cn: (i, 0)),
    )
    return pl.pallas_call(
        _pair_body,
        grid_spec=grid_spec,
        out_shape=jax.ShapeDtypeStruct((N, NF), jnp.float32),
        compiler_params=pltpu.CompilerParams(
            dimension_semantics=("parallel",)),
    )(cstart, cnum, A, B, batf, xs, w1, b1, w2, b2, offs)


def _node_body(h_ref, ag_ref, w2_ref, b2_ref, lw_ref, lb_ref, o_ref):
    x = jax.lax.dot(ag_ref[...], w2_ref[...], precision=HI) + b2_ref[...]
    x = _ssp(x)
    x = jax.lax.dot(x, lw_ref[...], precision=HI) + lb_ref[...]
    o_ref[...] = h_ref[...] + x


def _node(h, aggr, w2, b2, lw, lb):
    return pl.pallas_call(
        _node_body,
        grid=(N // RB,),
        in_specs=[
            pl.BlockSpec((RB, HIDDEN), lambda i: (i, 0)),
            pl.BlockSpec((RB, NF), lambda i: (i, 0)),
            pl.BlockSpec((NF, HIDDEN), lambda i: (0, 0)),
            pl.BlockSpec((1, HIDDEN), lambda i: (0, 0)),
            pl.BlockSpec((HIDDEN, HIDDEN), lambda i: (0, 0)),
            pl.BlockSpec((1, HIDDEN), lambda i: (0, 0)),
        ],
        out_specs=pl.BlockSpec((RB, HIDDEN), lambda i: (i, 0)),
        out_shape=jax.ShapeDtypeStruct((N, HIDDEN), jnp.float32),
    )(h, aggr, w2, b2, lw, lb)


def _pool_body(bat_ref, h_ref, sum_ref, cnt_ref):
    i = pl.program_id(0)

    @pl.when(i == 0)
    def _():
        sum_ref[...] = jnp.zeros_like(sum_ref)
        cnt_ref[...] = jnp.zeros_like(cnt_ref)

    brow = bat_ref[...].reshape(1, RB)
    mol = jax.lax.broadcasted_iota(jnp.int32, (NMOL, RB), 0).astype(jnp.float32)
    mt = (mol == brow).astype(jnp.float32)
    sum_ref[...] += jax.lax.dot(mt, h_ref[...], precision=HI)
    cnt_ref[...] += jnp.sum(mt, axis=1, keepdims=True)


def _pool(batf, h):
    return pl.pallas_call(
        _pool_body,
        grid=(N // RB,),
        in_specs=[
            pl.BlockSpec((RB, 1), lambda i: (i, 0)),
            pl.BlockSpec((RB, HIDDEN), lambda i: (i, 0)),
        ],
        out_specs=[
            pl.BlockSpec((NMOL, HIDDEN), lambda i: (0, 0)),
            pl.BlockSpec((NMOL, 1), lambda i: (0, 0)),
        ],
        out_shape=[
            jax.ShapeDtypeStruct((NMOL, HIDDEN), jnp.float32),
            jax.ShapeDtypeStruct((NMOL, 1), jnp.float32),
        ],
    )(batf, h)


def _head_body(s_ref, c_ref, w1_ref, b1_ref, w2_ref, b2_ref, o_ref):
    g = s_ref[...] / jnp.maximum(c_ref[...], 1.0)
    z1 = jnp.maximum(jax.lax.dot(g, w1_ref[...], precision=HI) + b1_ref[...], 0.0)
    o_ref[...] = jax.lax.dot(z1, w2_ref[...], precision=HI) + b2_ref[...]


def _head(sums, cnts, w1, b1, w2, b2):
    return pl.pallas_call(
        _head_body,
        in_specs=[
            pl.BlockSpec((NMOL, HIDDEN), lambda: (0, 0)),
            pl.BlockSpec((NMOL, 1), lambda: (0, 0)),
            pl.BlockSpec((HIDDEN, HIDDEN), lambda: (0, 0)),
            pl.BlockSpec((1, HIDDEN), lambda: (0, 0)),
            pl.BlockSpec((HIDDEN, NT), lambda: (0, 0)),
            pl.BlockSpec((1, NT), lambda: (0, 0)),
        ],
        out_specs=pl.BlockSpec((NMOL, NT), lambda: (0, 0)),
        out_shape=jax.ShapeDtypeStruct((NMOL, NT), jnp.float32),
    )(sums, cnts, w1, b1, w2, b2)


def kernel(z, pos, batch, emb, mlp_w1, mlp_b1, mlp_w2, mlp_b2, lin1_w,
           lin2_w, lin2_b, lin_w, lin_b, cls_w1, cls_b1, cls_w2, cls_b2):
    pos = pos.astype(jnp.float32)
    x2 = jnp.sum(pos * pos, axis=1, keepdims=True)
    one = jnp.ones((N, 1), jnp.float32)
    zero3 = jnp.zeros((N, 3), jnp.float32)
    A = jnp.concatenate([-2.0 * pos, x2, one, zero3], axis=1)
    B = jnp.concatenate([pos, one, x2, zero3], axis=1)
    batf = batch.astype(jnp.float32).reshape(N, 1)

    r0s = jnp.arange(NRB, dtype=jnp.int32) * TR
    firstmol = batch[r0s]
    lastmol = batch[r0s + TR - 1]
    jmin = jnp.searchsorted(batch, firstmol, side="left").astype(jnp.int32)
    jmax = jnp.searchsorted(batch, lastmol, side="right").astype(jnp.int32)
    cstart = jmin // TC
    cnum = (jmax - 1) // TC - cstart + 1

    offs_pad = np.full((1, NGP), 1e6, np.float32)
    offs_pad[0, :NG] = _OFFS_NP
    offs = jnp.asarray(offs_pad)

    h = _embed(z.astype(jnp.int32), emb)
    for i in range(NI):
        w1p = (jnp.zeros((NGP, NF), jnp.float32).at[:NG].set(mlp_w1[i])
               .astype(jnp.bfloat16))
        xs = _xs(h, lin1_w[i])
        aggr = _pair(cstart, cnum, A, B, batf, xs,
                     w1p, mlp_b1[i].reshape(1, NF),
                     mlp_w2[i].astype(jnp.bfloat16),
                     mlp_b2[i].reshape(1, NF), offs)
        h = _node(h, aggr, lin2_w[i], lin2_b[i].reshape(1, HIDDEN),
                  lin_w[i], lin_b[i].reshape(1, HIDDEN))

    sums, cnts = _pool(batf, h)
    return _head(sums, cnts, cls_w1, cls_b1.reshape(1, HIDDEN),
                 cls_w2, cls_b2.reshape(1, NT))


# speedup vs baseline: 1.0216x; 1.0216x over previous
import math

import jax
import jax.numpy as jnp
import numpy as np
from jax.experimental import pallas as pl
from jax.experimental.pallas import tpu as pltpu

N = 8192
NMOL = 256
HIDDEN = 128
NF = 128
NI = 6
NG = 50
NGP = 128
CUTOFF = 10.0
NT = 12
TR = 32
TC = 32
TRS = 4
NRB = N // TR
RB = 128
NZ = 100
LOG2 = math.log(2.0)
HI = jax.lax.Precision.HIGHEST
_OFFS_NP = np.linspace(0.0, CUTOFF, NG).astype(np.float32)
_DELTA = _OFFS_NP[1] - _OFFS_NP[0]
_COEFF = float(np.float32(-0.5) / (_DELTA * _DELTA))


def _ssp(x):
    return jnp.maximum(x, 0.0) + jnp.log1p(jnp.exp(-jnp.abs(x))) - LOG2


def _embed_body(z_ref, emb_ref, o_ref):
    z = z_ref[...]
    oh = (z == jax.lax.broadcasted_iota(jnp.int32, (RB, NZ), 1)).astype(jnp.float32)
    o_ref[...] = jax.lax.dot(oh, emb_ref[...], precision=HI)


def _embed(z, emb):
    return pl.pallas_call(
        _embed_body,
        grid=(N // RB,),
        in_specs=[
            pl.BlockSpec((RB, 1), lambda i: (i, 0)),
            pl.BlockSpec((NZ, HIDDEN), lambda i: (0, 0)),
        ],
        out_specs=pl.BlockSpec((RB, HIDDEN), lambda i: (i, 0)),
        out_shape=jax.ShapeDtypeStruct((N, HIDDEN), jnp.float32),
    )(z.reshape(N, 1), emb)


def _mm_body(x_ref, w_ref, o_ref):
    o_ref[...] = jax.lax.dot(x_ref[...], w_ref[...], precision=HI)


def _xs(h, w):
    return pl.pallas_call(
        _mm_body,
        grid=(N // RB,),
        in_specs=[
            pl.BlockSpec((RB, HIDDEN), lambda i: (i, 0)),
            pl.BlockSpec((HIDDEN, NF), lambda i: (0, 0)),
        ],
        out_specs=pl.BlockSpec((RB, NF), lambda i: (i, 0)),
        out_shape=jax.ShapeDtypeStruct((N, NF), jnp.float32),
    )(h, w)


_K = float(np.sqrt(np.float64(-_COEFF)))


def _sspl(x):
    return jnp.maximum(x, 0.0) + jnp.log1p(jnp.exp(-jnp.abs(x)))


def _pair_body(cs_ref, cn_ref, a_ref, b_ref, bat_ref, xs_ref,
               w1_ref, b1_ref, w2_ref, b2_ref, offs_ref, o_ref):
    i = pl.program_id(0)
    offs = offs_ref[...].reshape(1, 1, NGP)
    w1 = w1_ref[...]
    b1 = b1_ref[...]
    w2 = w2_ref[...]
    b2 = b2_ref[...]

    for s in range(TRS):
        sb = i * TRS + s
        r0 = sb * TR
        a_blk = a_ref[pl.ds(s * TR, TR), :]
        bcol = bat_ref[pl.ds(r0, TR), :]
        row_ids = r0 + jax.lax.broadcasted_iota(jnp.int32, (TR, TC), 0)

        def body(t, acc):
            c0 = t * TC
            b_j = b_ref[pl.ds(c0, TC), :]
            d2 = jax.lax.dot_general(
                a_blk, b_j, (((1,), (1,)), ((), ())), precision=HI)
            brow = bat_ref[pl.ds(c0, TC), :].reshape(1, TC)
            col_ids = c0 + jax.lax.broadcasted_iota(jnp.int32, (TR, TC), 1)
            mask = (bcol == brow) & (d2 <= CUTOFF * CUTOFF) & (row_ids != col_ids)
            dm = jnp.where(mask, d2, 1e9)
            w = jnp.sqrt(jnp.maximum(dm, 0.0))
            cc = jnp.where(dm < 1e8,
                           0.5 * (jnp.cos(w * (math.pi / CUTOFF)) + 1.0), 0.0)
            ws3 = (w * _K).reshape(TR, TC, 1)
            ea = jnp.exp(-(ws3 - offs) ** 2)
            ea = ea.astype(jnp.bfloat16).reshape(TR * TC, NGP)
            a1 = jax.lax.dot(ea, w1, preferred_element_type=jnp.float32) + b1
            wf = jax.lax.dot(_sspl(a1).astype(jnp.bfloat16), w2,
                             preferred_element_type=jnp.float32) + b2
            wf3 = wf.reshape(TR, TC, NF) * cc.reshape(TR, TC, 1)
            xsj = xs_ref[pl.ds(c0, TC), :]
            msg = wf3 * xsj[None, :, :]
            return acc + jnp.sum(msg, axis=1)

        t0 = cs_ref[sb]
        acc = jax.lax.fori_loop(t0, t0 + cn_ref[sb], body,
                                jnp.zeros((TR, NF), jnp.float32))
        o_ref[pl.ds(s * TR, TR), :] = acc


def _pair(cstart, cnum, A, B, batf, xs, w1, b1, w2, b2, offs):
    grid_spec = pltpu.PrefetchScalarGridSpec(
        num_scalar_prefetch=2,
        grid=(NRB // TRS,),
        in_specs=[
            pl.BlockSpec((TRS * TR, 8), lambda i, cs, cn: (i, 0)),
            pl.BlockSpec((N, 8), lambda i, cs, cn: (0, 0)),
            pl.BlockSpec((N, 1), lambda i, cs, cn: (0, 0)),
            pl.BlockSpec((N, NF), lambda i, cs, cn: (0, 0)),
            pl.BlockSpec((NGP, NF), lambda i, cs, cn: (0, 0)),
            pl.BlockSpec((1, NF), lambda i, cs, cn: (0, 0)),
            pl.BlockSpec((NF, NF), lambda i, cs, cn: (0, 0)),
            pl.BlockSpec((1, NF), lambda i, cs, cn: (0, 0)),
            pl.BlockSpec((1, NGP), lambda i, cs, cn: (0, 0)),
        ],
        out_specs=pl.BlockSpec((TRS * TR, NF), lambda i, cs, cn: (i, 0)),
    )
    return pl.pallas_call(
        _pair_body,
        grid_spec=grid_spec,
        out_shape=jax.ShapeDtypeStruct((N, NF), jnp.float32),
        compiler_params=pltpu.CompilerParams(
            dimension_semantics=("arbitrary",)),
    )(cstart, cnum, A, B, batf, xs, w1, b1, w2, b2, offs)


def _node_body(h_ref, ag_ref, w2_ref, b2_ref, lw_ref, lb_ref, o_ref):
    x = jax.lax.dot(ag_ref[...], w2_ref[...], precision=HI) + b2_ref[...]
    x = _ssp(x)
    x = jax.lax.dot(x, lw_ref[...], precision=HI) + lb_ref[...]
    o_ref[...] = h_ref[...] + x


def _node(h, aggr, w2, b2, lw, lb):
    return pl.pallas_call(
        _node_body,
        grid=(N // RB,),
        in_specs=[
            pl.BlockSpec((RB, HIDDEN), lambda i: (i, 0)),
            pl.BlockSpec((RB, NF), lambda i: (i, 0)),
            pl.BlockSpec((NF, HIDDEN), lambda i: (0, 0)),
            pl.BlockSpec((1, HIDDEN), lambda i: (0, 0)),
            pl.BlockSpec((HIDDEN, HIDDEN), lambda i: (0, 0)),
            pl.BlockSpec((1, HIDDEN), lambda i: (0, 0)),
        ],
        out_specs=pl.BlockSpec((RB, HIDDEN), lambda i: (i, 0)),
        out_shape=jax.ShapeDtypeStruct((N, HIDDEN), jnp.float32),
    )(h, aggr, w2, b2, lw, lb)


def _pool_body(bat_ref, h_ref, sum_ref, cnt_ref):
    i = pl.program_id(0)

    @pl.when(i == 0)
    def _():
        sum_ref[...] = jnp.zeros_like(sum_ref)
        cnt_ref[...] = jnp.zeros_like(cnt_ref)

    brow = bat_ref[...].reshape(1, RB)
    mol = jax.lax.broadcasted_iota(jnp.int32, (NMOL, RB), 0).astype(jnp.float32)
    mt = (mol == brow).astype(jnp.float32)
    sum_ref[...] += jax.lax.dot(mt, h_ref[...], precision=HI)
    cnt_ref[...] += jnp.sum(mt, axis=1, keepdims=True)


def _pool(batf, h):
    return pl.pallas_call(
        _pool_body,
        grid=(N // RB,),
        in_specs=[
            pl.BlockSpec((RB, 1), lambda i: (i, 0)),
            pl.BlockSpec((RB, HIDDEN), lambda i: (i, 0)),
        ],
        out_specs=[
            pl.BlockSpec((NMOL, HIDDEN), lambda i: (0, 0)),
            pl.BlockSpec((NMOL, 1), lambda i: (0, 0)),
        ],
        out_shape=[
            jax.ShapeDtypeStruct((NMOL, HIDDEN), jnp.float32),
            jax.ShapeDtypeStruct((NMOL, 1), jnp.float32),
        ],
    )(batf, h)


def _head_body(s_ref, c_ref, w1_ref, b1_ref, w2_ref, b2_ref, o_ref):
    g = s_ref[...] / jnp.maximum(c_ref[...], 1.0)
    z1 = jnp.maximum(jax.lax.dot(g, w1_ref[...], precision=HI) + b1_ref[...], 0.0)
    o_ref[...] = jax.lax.dot(z1, w2_ref[...], precision=HI) + b2_ref[...]


def _head(sums, cnts, w1, b1, w2, b2):
    return pl.pallas_call(
        _head_body,
        in_specs=[
            pl.BlockSpec((NMOL, HIDDEN), lambda: (0, 0)),
            pl.BlockSpec((NMOL, 1), lambda: (0, 0)),
            pl.BlockSpec((HIDDEN, HIDDEN), lambda: (0, 0)),
            pl.BlockSpec((1, HIDDEN), lambda: (0, 0)),
            pl.BlockSpec((HIDDEN, NT), lambda: (0, 0)),
            pl.BlockSpec((1, NT), lambda: (0, 0)),
        ],
        out_specs=pl.BlockSpec((NMOL, NT), lambda: (0, 0)),
        out_shape=jax.ShapeDtypeStruct((NMOL, NT), jnp.float32),
    )(sums, cnts, w1, b1, w2, b2)


def kernel(z, pos, batch, emb, mlp_w1, mlp_b1, mlp_w2, mlp_b2, lin1_w,
           lin2_w, lin2_b, lin_w, lin_b, cls_w1, cls_b1, cls_w2, cls_b2):
    pos = pos.astype(jnp.float32)
    x2 = jnp.sum(pos * pos, axis=1, keepdims=True)
    one = jnp.ones((N, 1), jnp.float32)
    zero3 = jnp.zeros((N, 3), jnp.float32)
    A = jnp.concatenate([-2.0 * pos, x2, one, zero3], axis=1)
    B = jnp.concatenate([pos, one, x2, zero3], axis=1)
    batf = batch.astype(jnp.float32).reshape(N, 1)

    r0s = jnp.arange(NRB, dtype=jnp.int32) * TR
    firstmol = batch[r0s]
    lastmol = batch[r0s + TR - 1]
    jmin = jnp.searchsorted(batch, firstmol, side="left").astype(jnp.int32)
    jmax = jnp.searchsorted(batch, lastmol, side="right").astype(jnp.int32)
    cstart = jmin // TC
    cnum = (jmax - 1) // TC - cstart + 1

    offs_pad = np.full((1, NGP), 1e6, np.float32) * np.float32(_K)
    offs_pad[0, :NG] = _OFFS_NP * np.float32(_K)
    offs = jnp.asarray(offs_pad)

    h = _embed(z.astype(jnp.int32), emb)
    for i in range(NI):
        w1p = (jnp.zeros((NGP, NF), jnp.float32).at[:NG].set(mlp_w1[i])
               .astype(jnp.bfloat16))
        xs = _xs(h, lin1_w[i])
        w2b = mlp_w2[i].astype(jnp.bfloat16)
        b2f = (mlp_b2[i] - LOG2 * jnp.sum(w2b.astype(jnp.float32), axis=0))
        aggr = _pair(cstart, cnum, A, B, batf, xs,
                     w1p, mlp_b1[i].reshape(1, NF),
                     w2b, b2f.reshape(1, NF), offs)
        h = _node(h, aggr, lin2_w[i], lin2_b[i].reshape(1, HIDDEN),
                  lin_w[i], lin_b[i].reshape(1, HIDDEN))

    sums, cnts = _pool(batf, h)
    return _head(sums, cnts, cls_w1, cls_b1.reshape(1, HIDDEN),
                 cls_w2, cls_b2.reshape(1, NT))


# speedup vs baseline: 1.0275x; 1.0058x over previous
import math

import jax
import jax.numpy as jnp
import numpy as np
from jax.experimental import pallas as pl
from jax.experimental.pallas import tpu as pltpu
from jax.experimental.pallas import tpu_sc as plsc

N = 8192
NMOL = 256
HIDDEN = 128
NF = 128
NI = 6
NG = 50
NGP = 128
CUTOFF = 10.0
NT = 12
TR = 32
TC = 32
TRS = 4
NRB = N // TR
RB = 128
NZ = 100
LOG2 = math.log(2.0)
HI = jax.lax.Precision.HIGHEST
_OFFS_NP = np.linspace(0.0, CUTOFF, NG).astype(np.float32)
_DELTA = _OFFS_NP[1] - _OFFS_NP[0]
_COEFF = float(np.float32(-0.5) / (_DELTA * _DELTA))


def _ssp(x):
    return jnp.maximum(x, 0.0) + jnp.log1p(jnp.exp(-jnp.abs(x))) - LOG2


_GW = 128


def _embed_sc(z, emb):
    mesh = plsc.VectorSubcoreMesh(core_axis_name="c", subcore_axis_name="s")

    @pl.kernel(out_type=jax.ShapeDtypeStruct((N, HIDDEN), jnp.float32),
               mesh=mesh)
    def gather_kernel(emb_hbm, zi_hbm, o_hbm):
        def body(i_vmem, o_vmem):
            pltpu.sync_copy(emb_hbm.at[i_vmem.at[0]], o_vmem)

        pltpu.emit_pipeline(
            body,
            grid=(N // _GW,),
            in_specs=[pl.BlockSpec((1, _GW), index_map=lambda i: (0, i))],
            out_specs=[pl.BlockSpec((_GW, HIDDEN), index_map=lambda i: (i, 0))],
            core_axis_name="s",
            dimension_semantics=(pltpu.PARALLEL,),
        )(zi_hbm, o_hbm)

    return gather_kernel(emb.astype(jnp.float32), z.reshape(1, N))


def _mm_body(x_ref, w_ref, o_ref):
    o_ref[...] = jax.lax.dot(x_ref[...], w_ref[...], precision=HI)


def _xs(h, w):
    return pl.pallas_call(
        _mm_body,
        grid=(N // RB,),
        in_specs=[
            pl.BlockSpec((RB, HIDDEN), lambda i: (i, 0)),
            pl.BlockSpec((HIDDEN, NF), lambda i: (0, 0)),
        ],
        out_specs=pl.BlockSpec((RB, NF), lambda i: (i, 0)),
        out_shape=jax.ShapeDtypeStruct((N, NF), jnp.float32),
    )(h, w)


_K = float(np.sqrt(np.float64(-_COEFF)))


def _sspl(x):
    return jnp.maximum(x, 0.0) + jnp.log1p(jnp.exp(-jnp.abs(x)))


def _pair_body(cs_ref, cn_ref, a_ref, b_ref, bat_ref, xs_ref,
               w1_ref, b1_ref, w2_ref, b2_ref, offs_ref, o_ref):
    i = pl.program_id(0)
    offs = offs_ref[...].reshape(1, 1, NGP)
    w1 = w1_ref[...]
    b1 = b1_ref[...]
    w2 = w2_ref[...]
    b2 = b2_ref[...]

    for s in range(TRS):
        sb = i * TRS + s
        r0 = sb * TR
        a_blk = a_ref[pl.ds(s * TR, TR), :]
        bcol = bat_ref[pl.ds(r0, TR), :]
        row_ids = r0 + jax.lax.broadcasted_iota(jnp.int32, (TR, TC), 0)

        def body(t, acc):
            c0 = t * TC
            b_j = b_ref[pl.ds(c0, TC), :]
            d2 = jax.lax.dot_general(
                a_blk, b_j, (((1,), (1,)), ((), ())), precision=HI)
            brow = bat_ref[pl.ds(c0, TC), :].reshape(1, TC)
            col_ids = c0 + jax.lax.broadcasted_iota(jnp.int32, (TR, TC), 1)
            mask = (bcol == brow) & (d2 <= CUTOFF * CUTOFF) & (row_ids != col_ids)
            dm = jnp.where(mask, d2, 1e9)
            w = jnp.sqrt(jnp.maximum(dm, 0.0))
            cc = jnp.where(dm < 1e8,
                           0.5 * (jnp.cos(w * (math.pi / CUTOFF)) + 1.0), 0.0)
            ws3 = (w * _K).reshape(TR, TC, 1)
            ea = jnp.exp(-(ws3 - offs) ** 2)
            ea = ea.astype(jnp.bfloat16).reshape(TR * TC, NGP)
            a1 = jax.lax.dot(ea, w1, preferred_element_type=jnp.float32) + b1
            wf = jax.lax.dot(_sspl(a1).astype(jnp.bfloat16), w2,
                             preferred_element_type=jnp.float32) + b2
            wf3 = wf.reshape(TR, TC, NF) * cc.reshape(TR, TC, 1)
            xsj = xs_ref[pl.ds(c0, TC), :]
            msg = wf3 * xsj[None, :, :]
            return acc + jnp.sum(msg, axis=1)

        t0 = cs_ref[sb]
        acc = jax.lax.fori_loop(t0, t0 + cn_ref[sb], body,
                                jnp.zeros((TR, NF), jnp.float32))
        o_ref[pl.ds(s * TR, TR), :] = acc


def _pair(cstart, cnum, A, B, batf, xs, w1, b1, w2, b2, offs):
    grid_spec = pltpu.PrefetchScalarGridSpec(
        num_scalar_prefetch=2,
        grid=(NRB // TRS,),
        in_specs=[
            pl.BlockSpec((TRS * TR, 8), lambda i, cs, cn: (i, 0)),
            pl.BlockSpec((N, 8), lambda i, cs, cn: (0, 0)),
            pl.BlockSpec((N, 1), lambda i, cs, cn: (0, 0)),
            pl.BlockSpec((N, NF), lambda i, cs, cn: (0, 0)),
            pl.BlockSpec((NGP, NF), lambda i, cs, cn: (0, 0)),
            pl.BlockSpec((1, NF), lambda i, cs, cn: (0, 0)),
            pl.BlockSpec((NF, NF), lambda i, cs, cn: (0, 0)),
            pl.BlockSpec((1, NF), lambda i, cs, cn: (0, 0)),
            pl.BlockSpec((1, NGP), lambda i, cs, cn: (0, 0)),
        ],
        out_specs=pl.BlockSpec((TRS * TR, NF), lambda i, cs, cn: (i, 0)),
    )
    return pl.pallas_call(
        _pair_body,
        grid_spec=grid_spec,
        out_shape=jax.ShapeDtypeStruct((N, NF), jnp.float32),
        compiler_params=pltpu.CompilerParams(
            dimension_semantics=("arbitrary",)),
    )(cstart, cnum, A, B, batf, xs, w1, b1, w2, b2, offs)


def _node_body(h_ref, ag_ref, w2_ref, b2_ref, lw_ref, lb_ref, o_ref):
    x = jax.lax.dot(ag_ref[...], w2_ref[...], precision=HI) + b2_ref[...]
    x = _ssp(x)
    x = jax.lax.dot(x, lw_ref[...], precision=HI) + lb_ref[...]
    o_ref[...] = h_ref[...] + x


def _node(h, aggr, w2, b2, lw, lb):
    return pl.pallas_call(
        _node_body,
        grid=(N // RB,),
        in_specs=[
            pl.BlockSpec((RB, HIDDEN), lambda i: (i, 0)),
            pl.BlockSpec((RB, NF), lambda i: (i, 0)),
            pl.BlockSpec((NF, HIDDEN), lambda i: (0, 0)),
            pl.BlockSpec((1, HIDDEN), lambda i: (0, 0)),
            pl.BlockSpec((HIDDEN, HIDDEN), lambda i: (0, 0)),
            pl.BlockSpec((1, HIDDEN), lambda i: (0, 0)),
        ],
        out_specs=pl.BlockSpec((RB, HIDDEN), lambda i: (i, 0)),
        out_shape=jax.ShapeDtypeStruct((N, HIDDEN), jnp.float32),
    )(h, aggr, w2, b2, lw, lb)


def _pool_body(bat_ref, h_ref, sum_ref, cnt_ref):
    i = pl.program_id(0)

    @pl.when(i == 0)
    def _():
        sum_ref[...] = jnp.zeros_like(sum_ref)
        cnt_ref[...] = jnp.zeros_like(cnt_ref)

    brow = bat_ref[...].reshape(1, RB)
    mol = jax.lax.broadcasted_iota(jnp.int32, (NMOL, RB), 0).astype(jnp.float32)
    mt = (mol == brow).astype(jnp.float32)
    sum_ref[...] += jax.lax.dot(mt, h_ref[...], precision=HI)
    cnt_ref[...] += jnp.sum(mt, axis=1, keepdims=True)


def _pool(batf, h):
    return pl.pallas_call(
        _pool_body,
        grid=(N // RB,),
        in_specs=[
            pl.BlockSpec((RB, 1), lambda i: (i, 0)),
            pl.BlockSpec((RB, HIDDEN), lambda i: (i, 0)),
        ],
        out_specs=[
            pl.BlockSpec((NMOL, HIDDEN), lambda i: (0, 0)),
            pl.BlockSpec((NMOL, 1), lambda i: (0, 0)),
        ],
        out_shape=[
            jax.ShapeDtypeStruct((NMOL, HIDDEN), jnp.float32),
            jax.ShapeDtypeStruct((NMOL, 1), jnp.float32),
        ],
    )(batf, h)


def _head_body(s_ref, c_ref, w1_ref, b1_ref, w2_ref, b2_ref, o_ref):
    g = s_ref[...] / jnp.maximum(c_ref[...], 1.0)
    z1 = jnp.maximum(jax.lax.dot(g, w1_ref[...], precision=HI) + b1_ref[...], 0.0)
    o_ref[...] = jax.lax.dot(z1, w2_ref[...], precision=HI) + b2_ref[...]


def _head(sums, cnts, w1, b1, w2, b2):
    return pl.pallas_call(
        _head_body,
        in_specs=[
            pl.BlockSpec((NMOL, HIDDEN), lambda: (0, 0)),
            pl.BlockSpec((NMOL, 1), lambda: (0, 0)),
            pl.BlockSpec((HIDDEN, HIDDEN), lambda: (0, 0)),
            pl.BlockSpec((1, HIDDEN), lambda: (0, 0)),
            pl.BlockSpec((HIDDEN, NT), lambda: (0, 0)),
            pl.BlockSpec((1, NT), lambda: (0, 0)),
        ],
        out_specs=pl.BlockSpec((NMOL, NT), lambda: (0, 0)),
        out_shape=jax.ShapeDtypeStruct((NMOL, NT), jnp.float32),
    )(sums, cnts, w1, b1, w2, b2)


def kernel(z, pos, batch, emb, mlp_w1, mlp_b1, mlp_w2, mlp_b2, lin1_w,
           lin2_w, lin2_b, lin_w, lin_b, cls_w1, cls_b1, cls_w2, cls_b2):
    pos = pos.astype(jnp.float32)
    x2 = jnp.sum(pos * pos, axis=1, keepdims=True)
    one = jnp.ones((N, 1), jnp.float32)
    zero3 = jnp.zeros((N, 3), jnp.float32)
    A = jnp.concatenate([-2.0 * pos, x2, one, zero3], axis=1)
    B = jnp.concatenate([pos, one, x2, zero3], axis=1)
    batf = batch.astype(jnp.float32).reshape(N, 1)

    r0s = jnp.arange(NRB, dtype=jnp.int32) * TR
    firstmol = batch[r0s]
    lastmol = batch[r0s + TR - 1]
    jmin = jnp.searchsorted(batch, firstmol, side="left").astype(jnp.int32)
    jmax = jnp.searchsorted(batch, lastmol, side="right").astype(jnp.int32)
    cstart = jmin // TC
    cnum = (jmax - 1) // TC - cstart + 1

    offs_pad = np.full((1, NGP), 1e6, np.float32) * np.float32(_K)
    offs_pad[0, :NG] = _OFFS_NP * np.float32(_K)
    offs = jnp.asarray(offs_pad)

    h = _embed_sc(z.astype(jnp.int32), emb)
    for i in range(NI):
        w1p = (jnp.zeros((NGP, NF), jnp.float32).at[:NG].set(mlp_w1[i])
               .astype(jnp.bfloat16))
        xs = _xs(h, lin1_w[i])
        w2b = mlp_w2[i].astype(jnp.bfloat16)
        b2f = (mlp_b2[i] - LOG2 * jnp.sum(w2b.astype(jnp.float32), axis=0))
        aggr = _pair(cstart, cnum, A, B, batf, xs,
                     w1p, mlp_b1[i].reshape(1, NF),
                     w2b, b2f.reshape(1, NF), offs)
        h = _node(h, aggr, lin2_w[i], lin2_b[i].reshape(1, HIDDEN),
                  lin_w[i], lin_b[i].reshape(1, HIDDEN))

    sums, cnts = _pool(batf, h)
    return _head(sums, cnts, cls_w1, cls_b1.reshape(1, HIDDEN),
                 cls_w2, cls_b2.reshape(1, NT))


# speedup vs baseline: 1.0292x; 1.0017x over previous
import math

import jax
import jax.numpy as jnp
import numpy as np
from jax.experimental import pallas as pl
from jax.experimental.pallas import tpu as pltpu
from jax.experimental.pallas import tpu_sc as plsc

N = 8192
NMOL = 256
HIDDEN = 128
NF = 128
NI = 6
NG = 50
NGP = 128
CUTOFF = 10.0
NT = 12
TR = 32
TC = 32
TRS = 8
NRB = N // TR
RB = 128
NZ = 100
LOG2 = math.log(2.0)
HI = jax.lax.Precision.HIGHEST
_OFFS_NP = np.linspace(0.0, CUTOFF, NG).astype(np.float32)
_DELTA = _OFFS_NP[1] - _OFFS_NP[0]
_COEFF = float(np.float32(-0.5) / (_DELTA * _DELTA))


def _ssp(x):
    return jnp.maximum(x, 0.0) + jnp.log1p(jnp.exp(-jnp.abs(x))) - LOG2


_GW = 128


def _embed_sc(z, emb):
    mesh = plsc.VectorSubcoreMesh(core_axis_name="c", subcore_axis_name="s")

    @pl.kernel(out_type=jax.ShapeDtypeStruct((N, HIDDEN), jnp.float32),
               mesh=mesh)
    def gather_kernel(emb_hbm, zi_hbm, o_hbm):
        def body(i_vmem, o_vmem):
            pltpu.sync_copy(emb_hbm.at[i_vmem.at[0]], o_vmem)

        pltpu.emit_pipeline(
            body,
            grid=(N // _GW,),
            in_specs=[pl.BlockSpec((1, _GW), index_map=lambda i: (0, i))],
            out_specs=[pl.BlockSpec((_GW, HIDDEN), index_map=lambda i: (i, 0))],
            core_axis_name="s",
            dimension_semantics=(pltpu.PARALLEL,),
        )(zi_hbm, o_hbm)

    return gather_kernel(emb.astype(jnp.float32), z.reshape(1, N))


def _mm_body(x_ref, w_ref, o_ref):
    o_ref[...] = jax.lax.dot(x_ref[...], w_ref[...], precision=HI)


def _xs(h, w):
    return pl.pallas_call(
        _mm_body,
        grid=(N // RB,),
        in_specs=[
            pl.BlockSpec((RB, HIDDEN), lambda i: (i, 0)),
            pl.BlockSpec((HIDDEN, NF), lambda i: (0, 0)),
        ],
        out_specs=pl.BlockSpec((RB, NF), lambda i: (i, 0)),
        out_shape=jax.ShapeDtypeStruct((N, NF), jnp.float32),
    )(h, w)


_K = float(np.sqrt(np.float64(-_COEFF)))


def _sspl(x):
    return jnp.maximum(x, 0.0) + jnp.log1p(jnp.exp(-jnp.abs(x)))


def _pair_body(cs_ref, cn_ref, a_ref, b_ref, bat_ref, xs_ref,
               w1_ref, b1_ref, w2_ref, b2_ref, offs_ref, o_ref):
    i = pl.program_id(0)
    offs = offs_ref[...].reshape(1, 1, NGP)
    w1 = w1_ref[...]
    b1 = b1_ref[...]
    w2 = w2_ref[...]
    b2 = b2_ref[...]

    for s in range(TRS):
        sb = i * TRS + s
        r0 = sb * TR
        a_blk = a_ref[pl.ds(s * TR, TR), :]
        bcol = bat_ref[pl.ds(r0, TR), :]
        row_ids = r0 + jax.lax.broadcasted_iota(jnp.int32, (TR, TC), 0)

        def body(t, acc):
            c0 = t * TC
            b_j = b_ref[pl.ds(c0, TC), :]
            d2 = jax.lax.dot_general(
                a_blk, b_j, (((1,), (1,)), ((), ())), precision=HI)
            brow = bat_ref[pl.ds(c0, TC), :].reshape(1, TC)
            col_ids = c0 + jax.lax.broadcasted_iota(jnp.int32, (TR, TC), 1)
            mask = (bcol == brow) & (d2 <= CUTOFF * CUTOFF) & (row_ids != col_ids)
            dm = jnp.where(mask, d2, 1e9)
            w = jnp.sqrt(jnp.maximum(dm, 0.0))
            cc = jnp.where(dm < 1e8,
                           0.5 * (jnp.cos(w * (math.pi / CUTOFF)) + 1.0), 0.0)
            ws3 = (w * _K).reshape(TR, TC, 1)
            ea = jnp.exp(-(ws3 - offs) ** 2)
            ea = ea.astype(jnp.bfloat16).reshape(TR * TC, NGP)
            a1 = jax.lax.dot(ea, w1, preferred_element_type=jnp.float32) + b1
            wf = jax.lax.dot(_sspl(a1).astype(jnp.bfloat16), w2,
                             preferred_element_type=jnp.float32) + b2
            wf3 = wf.reshape(TR, TC, NF) * cc.reshape(TR, TC, 1)
            xsj = xs_ref[pl.ds(c0, TC), :]
            msg = wf3 * xsj[None, :, :]
            return acc + jnp.sum(msg, axis=1)

        t0 = cs_ref[sb]
        acc = jax.lax.fori_loop(t0, t0 + cn_ref[sb], body,
                                jnp.zeros((TR, NF), jnp.float32))
        o_ref[pl.ds(s * TR, TR), :] = acc


def _pair(cstart, cnum, A, B, batf, xs, w1, b1, w2, b2, offs):
    grid_spec = pltpu.PrefetchScalarGridSpec(
        num_scalar_prefetch=2,
        grid=(NRB // TRS,),
        in_specs=[
            pl.BlockSpec((TRS * TR, 8), lambda i, cs, cn: (i, 0)),
            pl.BlockSpec((N, 8), lambda i, cs, cn: (0, 0)),
            pl.BlockSpec((N, 1), lambda i, cs, cn: (0, 0)),
            pl.BlockSpec((N, NF), lambda i, cs, cn: (0, 0)),
            pl.BlockSpec((NGP, NF), lambda i, cs, cn: (0, 0)),
            pl.BlockSpec((1, NF), lambda i, cs, cn: (0, 0)),
            pl.BlockSpec((NF, NF), lambda i, cs, cn: (0, 0)),
            pl.BlockSpec((1, NF), lambda i, cs, cn: (0, 0)),
            pl.BlockSpec((1, NGP), lambda i, cs, cn: (0, 0)),
        ],
        out_specs=pl.BlockSpec((TRS * TR, NF), lambda i, cs, cn: (i, 0)),
    )
    return pl.pallas_call(
        _pair_body,
        grid_spec=grid_spec,
        out_shape=jax.ShapeDtypeStruct((N, NF), jnp.float32),
        compiler_params=pltpu.CompilerParams(
            dimension_semantics=("arbitrary",)),
    )(cstart, cnum, A, B, batf, xs, w1, b1, w2, b2, offs)


def _node_body(h_ref, ag_ref, w2_ref, b2_ref, lw_ref, lb_ref, o_ref):
    x = jax.lax.dot(ag_ref[...], w2_ref[...], precision=HI) + b2_ref[...]
    x = _ssp(x)
    x = jax.lax.dot(x, lw_ref[...], precision=HI) + lb_ref[...]
    o_ref[...] = h_ref[...] + x


def _node(h, aggr, w2, b2, lw, lb):
    return pl.pallas_call(
        _node_body,
        grid=(N // RB,),
        in_specs=[
            pl.BlockSpec((RB, HIDDEN), lambda i: (i, 0)),
            pl.BlockSpec((RB, NF), lambda i: (i, 0)),
            pl.BlockSpec((NF, HIDDEN), lambda i: (0, 0)),
            pl.BlockSpec((1, HIDDEN), lambda i: (0, 0)),
            pl.BlockSpec((HIDDEN, HIDDEN), lambda i: (0, 0)),
            pl.BlockSpec((1, HIDDEN), lambda i: (0, 0)),
        ],
        out_specs=pl.BlockSpec((RB, HIDDEN), lambda i: (i, 0)),
        out_shape=jax.ShapeDtypeStruct((N, HIDDEN), jnp.float32),
    )(h, aggr, w2, b2, lw, lb)


def _pool_body(bat_ref, h_ref, sum_ref, cnt_ref):
    i = pl.program_id(0)

    @pl.when(i == 0)
    def _():
        sum_ref[...] = jnp.zeros_like(sum_ref)
        cnt_ref[...] = jnp.zeros_like(cnt_ref)

    brow = bat_ref[...].reshape(1, RB)
    mol = jax.lax.broadcasted_iota(jnp.int32, (NMOL, RB), 0).astype(jnp.float32)
    mt = (mol == brow).astype(jnp.float32)
    sum_ref[...] += jax.lax.dot(mt, h_ref[...], precision=HI)
    cnt_ref[...] += jnp.sum(mt, axis=1, keepdims=True)


def _pool(batf, h):
    return pl.pallas_call(
        _pool_body,
        grid=(N // RB,),
        in_specs=[
            pl.BlockSpec((RB, 1), lambda i: (i, 0)),
            pl.BlockSpec((RB, HIDDEN), lambda i: (i, 0)),
        ],
        out_specs=[
            pl.BlockSpec((NMOL, HIDDEN), lambda i: (0, 0)),
            pl.BlockSpec((NMOL, 1), lambda i: (0, 0)),
        ],
        out_shape=[
            jax.ShapeDtypeStruct((NMOL, HIDDEN), jnp.float32),
            jax.ShapeDtypeStruct((NMOL, 1), jnp.float32),
        ],
    )(batf, h)


def _head_body(s_ref, c_ref, w1_ref, b1_ref, w2_ref, b2_ref, o_ref):
    g = s_ref[...] / jnp.maximum(c_ref[...], 1.0)
    z1 = jnp.maximum(jax.lax.dot(g, w1_ref[...], precision=HI) + b1_ref[...], 0.0)
    o_ref[...] = jax.lax.dot(z1, w2_ref[...], precision=HI) + b2_ref[...]


def _head(sums, cnts, w1, b1, w2, b2):
    return pl.pallas_call(
        _head_body,
        in_specs=[
            pl.BlockSpec((NMOL, HIDDEN), lambda: (0, 0)),
            pl.BlockSpec((NMOL, 1), lambda: (0, 0)),
            pl.BlockSpec((HIDDEN, HIDDEN), lambda: (0, 0)),
            pl.BlockSpec((1, HIDDEN), lambda: (0, 0)),
            pl.BlockSpec((HIDDEN, NT), lambda: (0, 0)),
            pl.BlockSpec((1, NT), lambda: (0, 0)),
        ],
        out_specs=pl.BlockSpec((NMOL, NT), lambda: (0, 0)),
        out_shape=jax.ShapeDtypeStruct((NMOL, NT), jnp.float32),
    )(sums, cnts, w1, b1, w2, b2)


def kernel(z, pos, batch, emb, mlp_w1, mlp_b1, mlp_w2, mlp_b2, lin1_w,
           lin2_w, lin2_b, lin_w, lin_b, cls_w1, cls_b1, cls_w2, cls_b2):
    pos = pos.astype(jnp.float32)
    x2 = jnp.sum(pos * pos, axis=1, keepdims=True)
    one = jnp.ones((N, 1), jnp.float32)
    zero3 = jnp.zeros((N, 3), jnp.float32)
    A = jnp.concatenate([-2.0 * pos, x2, one, zero3], axis=1)
    B = jnp.concatenate([pos, one, x2, zero3], axis=1)
    batf = batch.astype(jnp.float32).reshape(N, 1)

    r0s = jnp.arange(NRB, dtype=jnp.int32) * TR
    firstmol = batch[r0s]
    lastmol = batch[r0s + TR - 1]
    jmin = jnp.searchsorted(batch, firstmol, side="left").astype(jnp.int32)
    jmax = jnp.searchsorted(batch, lastmol, side="right").astype(jnp.int32)
    cstart = jmin // TC
    cnum = (jmax - 1) // TC - cstart + 1

    offs_pad = np.full((1, NGP), 1e6, np.float32) * np.float32(_K)
    offs_pad[0, :NG] = _OFFS_NP * np.float32(_K)
    offs = jnp.asarray(offs_pad)

    h = _embed_sc(z.astype(jnp.int32), emb)
    for i in range(NI):
        w1p = (jnp.zeros((NGP, NF), jnp.float32).at[:NG].set(mlp_w1[i])
               .astype(jnp.bfloat16))
        xs = _xs(h, lin1_w[i])
        w2b = mlp_w2[i].astype(jnp.bfloat16)
        b2f = (mlp_b2[i] - LOG2 * jnp.sum(w2b.astype(jnp.float32), axis=0))
        aggr = _pair(cstart, cnum, A, B, batf, xs,
                     w1p, mlp_b1[i].reshape(1, NF),
                     w2b, b2f.reshape(1, NF), offs)
        h = _node(h, aggr, lin2_w[i], lin2_b[i].reshape(1, HIDDEN),
                  lin_w[i], lin_b[i].reshape(1, HIDDEN))

    sums, cnts = _pool(batf, h)
    return _head(sums, cnts, cls_w1, cls_b1.reshape(1, HIDDEN),
                 cls_w2, cls_b2.reshape(1, NT))


# speedup vs baseline: 1.3570x; 1.3185x over previous
import math

import jax
import jax.numpy as jnp
import numpy as np
from jax.experimental import pallas as pl
from jax.experimental.pallas import tpu as pltpu
from jax.experimental.pallas import tpu_sc as plsc

N = 8192
NMOL = 256
HIDDEN = 128
NF = 128
NI = 6
NG = 50
NGP = 128
CUTOFF = 10.0
NT = 12
TR = 32
TC = 32
TRS = 8
NRB = N // TR
RB = 128
NZ = 100
LOG2 = math.log(2.0)
HI = jax.lax.Precision.HIGHEST
_OFFS_NP = np.linspace(0.0, CUTOFF, NG).astype(np.float32)
_DELTA = _OFFS_NP[1] - _OFFS_NP[0]
_COEFF = float(np.float32(-0.5) / (_DELTA * _DELTA))


def _ssp(x):
    return jnp.maximum(x, 0.0) + jnp.log1p(jnp.exp(-jnp.abs(x))) - LOG2


_GW = 128


def _embed_sc(z, emb):
    mesh = plsc.VectorSubcoreMesh(core_axis_name="c", subcore_axis_name="s")

    @pl.kernel(out_type=jax.ShapeDtypeStruct((N, HIDDEN), jnp.float32),
               mesh=mesh)
    def gather_kernel(emb_hbm, zi_hbm, o_hbm):
        def body(i_vmem, o_vmem):
            pltpu.sync_copy(emb_hbm.at[i_vmem.at[0]], o_vmem)

        pltpu.emit_pipeline(
            body,
            grid=(N // _GW,),
            in_specs=[pl.BlockSpec((1, _GW), index_map=lambda i: (0, i))],
            out_specs=[pl.BlockSpec((_GW, HIDDEN), index_map=lambda i: (i, 0))],
            core_axis_name="s",
            dimension_semantics=(pltpu.PARALLEL,),
        )(zi_hbm, o_hbm)

    return gather_kernel(emb.astype(jnp.float32), z.reshape(1, N))


def _mm_body(x_ref, w_ref, o_ref):
    o_ref[...] = jax.lax.dot(x_ref[...], w_ref[...], precision=HI)


def _xs(h, w):
    return pl.pallas_call(
        _mm_body,
        grid=(N // RB,),
        in_specs=[
            pl.BlockSpec((RB, HIDDEN), lambda i: (i, 0)),
            pl.BlockSpec((HIDDEN, NF), lambda i: (0, 0)),
        ],
        out_specs=pl.BlockSpec((RB, NF), lambda i: (i, 0)),
        out_shape=jax.ShapeDtypeStruct((N, NF), jnp.float32),
    )(h, w)


_K = float(np.sqrt(np.float64(-_COEFF)))


def _sspl(x):
    return jnp.maximum(x, 0.0) + jnp.log1p(jnp.exp(-jnp.abs(x)))


def _pair_body(cs_ref, cn_ref, a_ref, b_ref, bat_ref, xs_ref,
               w1_ref, b1_ref, w2_ref, b2_ref, offs_ref, o_ref):
    i = pl.program_id(0)
    offs = offs_ref[...].reshape(1, 1, NGP)
    w1 = w1_ref[...]
    b1 = b1_ref[...]
    w2 = w2_ref[...]
    b2 = b2_ref[...]

    @pl.when(i == 0)
    def _():
        o_ref[...] = jnp.zeros_like(o_ref)

    for s in range(TRS):
        sb = i * TRS + s
        r0 = sb * TR
        a_blk = a_ref[pl.ds(s * TR, TR), :]
        bcol = bat_ref[pl.ds(r0, TR), :]
        row_ids = r0 + jax.lax.broadcasted_iota(jnp.int32, (TR, TC), 0)
        xsr = xs_ref[pl.ds(r0, TR), :]

        def body(t, acc):
            c0 = t * TC
            b_j = b_ref[pl.ds(c0, TC), :]
            d2 = jax.lax.dot_general(
                a_blk, b_j, (((1,), (1,)), ((), ())), precision=HI)
            brow = bat_ref[pl.ds(c0, TC), :].reshape(1, TC)
            col_ids = c0 + jax.lax.broadcasted_iota(jnp.int32, (TR, TC), 1)
            mask = (bcol == brow) & (d2 <= CUTOFF * CUTOFF) & (row_ids != col_ids)
            dm = jnp.where(mask, d2, 1e9)
            w = jnp.sqrt(jnp.maximum(dm, 0.0))
            cc = jnp.where(dm < 1e8,
                           0.5 * (jnp.cos(w * (math.pi / CUTOFF)) + 1.0), 0.0)
            ws3 = (w * _K).reshape(TR, TC, 1)
            ea = jnp.exp(-(ws3 - offs) ** 2)
            ea = ea.astype(jnp.bfloat16).reshape(TR * TC, NGP)
            a1 = jax.lax.dot(ea, w1, preferred_element_type=jnp.float32) + b1
            wf = jax.lax.dot(_sspl(a1).astype(jnp.bfloat16), w2,
                             preferred_element_type=jnp.float32) + b2
            wf3 = wf.reshape(TR, TC, NF) * cc.reshape(TR, TC, 1)
            xsj = xs_ref[pl.ds(c0, TC), :]
            msg = wf3 * xsj[None, :, :]

            @pl.when(t > sb)
            def _():
                rmsg = wf3 * xsr[:, None, :]
                o_ref[pl.ds(c0, TC), :] += jnp.sum(rmsg, axis=0)

            return acc + jnp.sum(msg, axis=1)

        t0 = jnp.maximum(cs_ref[sb], sb)
        acc = jax.lax.fori_loop(t0, cs_ref[sb] + cn_ref[sb], body,
                                jnp.zeros((TR, NF), jnp.float32))
        o_ref[pl.ds(r0, TR), :] += acc


def _pair(cstart, cnum, A, B, batf, xs, w1, b1, w2, b2, offs):
    grid_spec = pltpu.PrefetchScalarGridSpec(
        num_scalar_prefetch=2,
        grid=(NRB // TRS,),
        in_specs=[
            pl.BlockSpec((TRS * TR, 8), lambda i, cs, cn: (i, 0)),
            pl.BlockSpec((N, 8), lambda i, cs, cn: (0, 0)),
            pl.BlockSpec((N, 1), lambda i, cs, cn: (0, 0)),
            pl.BlockSpec((N, NF), lambda i, cs, cn: (0, 0)),
            pl.BlockSpec((NGP, NF), lambda i, cs, cn: (0, 0)),
            pl.BlockSpec((1, NF), lambda i, cs, cn: (0, 0)),
            pl.BlockSpec((NF, NF), lambda i, cs, cn: (0, 0)),
            pl.BlockSpec((1, NF), lambda i, cs, cn: (0, 0)),
            pl.BlockSpec((1, NGP), lambda i, cs, cn: (0, 0)),
        ],
        out_specs=pl.BlockSpec((N, NF), lambda i, cs, cn: (0, 0)),
    )
    return pl.pallas_call(
        _pair_body,
        grid_spec=grid_spec,
        out_shape=jax.ShapeDtypeStruct((N, NF), jnp.float32),
        compiler_params=pltpu.CompilerParams(
            dimension_semantics=("arbitrary",)),
    )(cstart, cnum, A, B, batf, xs, w1, b1, w2, b2, offs)


def _node_body(h_ref, ag_ref, w2_ref, b2_ref, lw_ref, lb_ref, o_ref):
    x = jax.lax.dot(ag_ref[...], w2_ref[...], precision=HI) + b2_ref[...]
    x = _ssp(x)
    x = jax.lax.dot(x, lw_ref[...], precision=HI) + lb_ref[...]
    o_ref[...] = h_ref[...] + x


def _node(h, aggr, w2, b2, lw, lb):
    return pl.pallas_call(
        _node_body,
        grid=(N // RB,),
        in_specs=[
            pl.BlockSpec((RB, HIDDEN), lambda i: (i, 0)),
            pl.BlockSpec((RB, NF), lambda i: (i, 0)),
            pl.BlockSpec((NF, HIDDEN), lambda i: (0, 0)),
            pl.BlockSpec((1, HIDDEN), lambda i: (0, 0)),
            pl.BlockSpec((HIDDEN, HIDDEN), lambda i: (0, 0)),
            pl.BlockSpec((1, HIDDEN), lambda i: (0, 0)),
        ],
        out_specs=pl.BlockSpec((RB, HIDDEN), lambda i: (i, 0)),
        out_shape=jax.ShapeDtypeStruct((N, HIDDEN), jnp.float32),
    )(h, aggr, w2, b2, lw, lb)


def _pool_body(bat_ref, h_ref, sum_ref, cnt_ref):
    i = pl.program_id(0)

    @pl.when(i == 0)
    def _():
        sum_ref[...] = jnp.zeros_like(sum_ref)
        cnt_ref[...] = jnp.zeros_like(cnt_ref)

    brow = bat_ref[...].reshape(1, RB)
    mol = jax.lax.broadcasted_iota(jnp.int32, (NMOL, RB), 0).astype(jnp.float32)
    mt = (mol == brow).astype(jnp.float32)
    sum_ref[...] += jax.lax.dot(mt, h_ref[...], precision=HI)
    cnt_ref[...] += jnp.sum(mt, axis=1, keepdims=True)


def _pool(batf, h):
    return pl.pallas_call(
        _pool_body,
        grid=(N // RB,),
        in_specs=[
            pl.BlockSpec((RB, 1), lambda i: (i, 0)),
            pl.BlockSpec((RB, HIDDEN), lambda i: (i, 0)),
        ],
        out_specs=[
            pl.BlockSpec((NMOL, HIDDEN), lambda i: (0, 0)),
            pl.BlockSpec((NMOL, 1), lambda i: (0, 0)),
        ],
        out_shape=[
            jax.ShapeDtypeStruct((NMOL, HIDDEN), jnp.float32),
            jax.ShapeDtypeStruct((NMOL, 1), jnp.float32),
        ],
    )(batf, h)


def _head_body(s_ref, c_ref, w1_ref, b1_ref, w2_ref, b2_ref, o_ref):
    g = s_ref[...] / jnp.maximum(c_ref[...], 1.0)
    z1 = jnp.maximum(jax.lax.dot(g, w1_ref[...], precision=HI) + b1_ref[...], 0.0)
    o_ref[...] = jax.lax.dot(z1, w2_ref[...], precision=HI) + b2_ref[...]


def _head(sums, cnts, w1, b1, w2, b2):
    return pl.pallas_call(
        _head_body,
        in_specs=[
            pl.BlockSpec((NMOL, HIDDEN), lambda: (0, 0)),
            pl.BlockSpec((NMOL, 1), lambda: (0, 0)),
            pl.BlockSpec((HIDDEN, HIDDEN), lambda: (0, 0)),
            pl.BlockSpec((1, HIDDEN), lambda: (0, 0)),
            pl.BlockSpec((HIDDEN, NT), lambda: (0, 0)),
            pl.BlockSpec((1, NT), lambda: (0, 0)),
        ],
        out_specs=pl.BlockSpec((NMOL, NT), lambda: (0, 0)),
        out_shape=jax.ShapeDtypeStruct((NMOL, NT), jnp.float32),
    )(sums, cnts, w1, b1, w2, b2)


def kernel(z, pos, batch, emb, mlp_w1, mlp_b1, mlp_w2, mlp_b2, lin1_w,
           lin2_w, lin2_b, lin_w, lin_b, cls_w1, cls_b1, cls_w2, cls_b2):
    pos = pos.astype(jnp.float32)
    x2 = jnp.sum(pos * pos, axis=1, keepdims=True)
    one = jnp.ones((N, 1), jnp.float32)
    zero3 = jnp.zeros((N, 3), jnp.float32)
    A = jnp.concatenate([-2.0 * pos, x2, one, zero3], axis=1)
    B = jnp.concatenate([pos, one, x2, zero3], axis=1)
    batf = batch.astype(jnp.float32).reshape(N, 1)

    r0s = jnp.arange(NRB, dtype=jnp.int32) * TR
    firstmol = batch[r0s]
    lastmol = batch[r0s + TR - 1]
    jmin = jnp.searchsorted(batch, firstmol, side="left").astype(jnp.int32)
    jmax = jnp.searchsorted(batch, lastmol, side="right").astype(jnp.int32)
    cstart = jmin // TC
    cnum = (jmax - 1) // TC - cstart + 1

    offs_pad = np.full((1, NGP), 1e6, np.float32) * np.float32(_K)
    offs_pad[0, :NG] = _OFFS_NP * np.float32(_K)
    offs = jnp.asarray(offs_pad)

    h = _embed_sc(z.astype(jnp.int32), emb)
    for i in range(NI):
        w1p = (jnp.zeros((NGP, NF), jnp.float32).at[:NG].set(mlp_w1[i])
               .astype(jnp.bfloat16))
        xs = _xs(h, lin1_w[i])
        w2b = mlp_w2[i].astype(jnp.bfloat16)
        b2f = (mlp_b2[i] - LOG2 * jnp.sum(w2b.astype(jnp.float32), axis=0))
        aggr = _pair(cstart, cnum, A, B, batf, xs,
                     w1p, mlp_b1[i].reshape(1, NF),
                     w2b, b2f.reshape(1, NF), offs)
        h = _node(h, aggr, lin2_w[i], lin2_b[i].reshape(1, HIDDEN),
                  lin_w[i], lin_b[i].reshape(1, HIDDEN))

    sums, cnts = _pool(batf, h)
    return _head(sums, cnts, cls_w1, cls_b1.reshape(1, HIDDEN),
                 cls_w2, cls_b2.reshape(1, NT))


# speedup vs baseline: 1.4138x; 1.0418x over previous
import math

import jax
import jax.numpy as jnp
import numpy as np
from jax.experimental import pallas as pl
from jax.experimental.pallas import tpu as pltpu
from jax.experimental.pallas import tpu_sc as plsc

N = 8192
NMOL = 256
HIDDEN = 128
NF = 128
NI = 6
NG = 50
NGP = 128
CUTOFF = 10.0
NT = 12
TR = 32
TC = 32
TRS = 8
NRB = N // TR
RB = 128
NZ = 100
LOG2 = math.log(2.0)
HI = jax.lax.Precision.HIGHEST
_OFFS_NP = np.linspace(0.0, CUTOFF, NG).astype(np.float32)
_DELTA = _OFFS_NP[1] - _OFFS_NP[0]
_COEFF = float(np.float32(-0.5) / (_DELTA * _DELTA))


def _ssp(x):
    return jnp.maximum(x, 0.0) + jnp.log1p(jnp.exp(-jnp.abs(x))) - LOG2


_GW = 128


def _embed_sc(z, emb):
    mesh = plsc.VectorSubcoreMesh(core_axis_name="c", subcore_axis_name="s")

    @pl.kernel(out_type=jax.ShapeDtypeStruct((N, HIDDEN), jnp.float32),
               mesh=mesh)
    def gather_kernel(emb_hbm, zi_hbm, o_hbm):
        def body(i_vmem, o_vmem):
            pltpu.sync_copy(emb_hbm.at[i_vmem.at[0]], o_vmem)

        pltpu.emit_pipeline(
            body,
            grid=(N // _GW,),
            in_specs=[pl.BlockSpec((1, _GW), index_map=lambda i: (0, i))],
            out_specs=[pl.BlockSpec((_GW, HIDDEN), index_map=lambda i: (i, 0))],
            core_axis_name="s",
            dimension_semantics=(pltpu.PARALLEL,),
        )(zi_hbm, o_hbm)

    return gather_kernel(emb.astype(jnp.float32), z.reshape(1, N))


def _mm_body(x_ref, w_ref, o_ref):
    o_ref[...] = jax.lax.dot(x_ref[...], w_ref[...], precision=HI)


def _xs(h, w):
    return pl.pallas_call(
        _mm_body,
        grid=(N // RB,),
        in_specs=[
            pl.BlockSpec((RB, HIDDEN), lambda i: (i, 0)),
            pl.BlockSpec((HIDDEN, NF), lambda i: (0, 0)),
        ],
        out_specs=pl.BlockSpec((RB, NF), lambda i: (i, 0)),
        out_shape=jax.ShapeDtypeStruct((N, NF), jnp.float32),
    )(h, w)


_K = float(np.sqrt(np.float64(-_COEFF)))


def _sspl(x):
    return jnp.maximum(x, 0.0) + jnp.log1p(jnp.exp(-jnp.abs(x)))


def _pair_body(cs_ref, cn_ref, a_ref, b_ref, bat_ref, xs_ref,
               w1_ref, b1_ref, w2_ref, b2_ref, offs_ref, o_ref):
    i = pl.program_id(0)
    offs = offs_ref[...].reshape(1, 1, NGP)
    w1 = w1_ref[...]
    b1 = b1_ref[...]
    w2 = w2_ref[...]
    b2 = b2_ref[...]

    @pl.when(i == 0)
    def _():
        o_ref[...] = jnp.zeros_like(o_ref)

    for s in range(TRS):
        sb = i * TRS + s
        r0 = sb * TR
        a_blk = a_ref[pl.ds(s * TR, TR), :]
        bcol = bat_ref[pl.ds(r0, TR), :]
        row_ids = r0 + jax.lax.broadcasted_iota(jnp.int32, (TR, TC), 0)
        xsr = xs_ref[pl.ds(r0, TR), :]

        def body(t, acc):
            c0 = t * TC
            b_j = b_ref[pl.ds(c0, TC), :]
            d2 = jax.lax.dot_general(
                a_blk, b_j, (((1,), (1,)), ((), ())), precision=HI)
            brow = bat_ref[pl.ds(c0, TC), :].reshape(1, TC)
            col_ids = c0 + jax.lax.broadcasted_iota(jnp.int32, (TR, TC), 1)
            mask = (bcol == brow) & (d2 <= CUTOFF * CUTOFF) & (row_ids != col_ids)
            dm = jnp.where(mask, d2, 1e9)
            w = jnp.sqrt(jnp.maximum(dm, 0.0))
            cc = jnp.where(dm < 1e8,
                           0.5 * (jnp.cos(w * (math.pi / CUTOFF)) + 1.0), 0.0)
            ws3 = (w * _K).reshape(TR, TC, 1)
            ea = jnp.exp(-(ws3 - offs) ** 2)
            ea = ea.astype(jnp.bfloat16).reshape(TR * TC, NGP)
            a1 = jax.lax.dot(ea, w1, preferred_element_type=jnp.float32) + b1
            wf = jax.lax.dot(_sspl(a1).astype(jnp.bfloat16), w2,
                             preferred_element_type=jnp.float32) + b2
            wf3 = wf.reshape(TR, TC, NF) * cc.reshape(TR, TC, 1)
            xsj = xs_ref[pl.ds(c0, TC), :]
            msg = wf3 * xsj[None, :, :]

            @pl.when(t > sb)
            def _():
                rmsg = wf3 * xsr[:, None, :]
                o_ref[pl.ds(c0, TC), :] += jnp.sum(rmsg, axis=0)

            return acc + jnp.sum(msg, axis=1)

        t0 = jnp.maximum(cs_ref[sb], sb)
        acc = jax.lax.fori_loop(t0, cs_ref[sb] + cn_ref[sb], body,
                                jnp.zeros((TR, NF), jnp.float32))
        o_ref[pl.ds(r0, TR), :] += acc


def _pair(cstart, cnum, A, B, batf, xs, w1, b1, w2, b2, offs):
    grid_spec = pltpu.PrefetchScalarGridSpec(
        num_scalar_prefetch=2,
        grid=(NRB // TRS,),
        in_specs=[
            pl.BlockSpec((TRS * TR, 8), lambda i, cs, cn: (i, 0)),
            pl.BlockSpec((N, 8), lambda i, cs, cn: (0, 0)),
            pl.BlockSpec((N, 1), lambda i, cs, cn: (0, 0)),
            pl.BlockSpec((N, NF), lambda i, cs, cn: (0, 0)),
            pl.BlockSpec((NGP, NF), lambda i, cs, cn: (0, 0)),
            pl.BlockSpec((1, NF), lambda i, cs, cn: (0, 0)),
            pl.BlockSpec((NF, NF), lambda i, cs, cn: (0, 0)),
            pl.BlockSpec((1, NF), lambda i, cs, cn: (0, 0)),
            pl.BlockSpec((1, NGP), lambda i, cs, cn: (0, 0)),
        ],
        out_specs=pl.BlockSpec((N, NF), lambda i, cs, cn: (0, 0)),
    )
    return pl.pallas_call(
        _pair_body,
        grid_spec=grid_spec,
        out_shape=jax.ShapeDtypeStruct((N, NF), jnp.float32),
        compiler_params=pltpu.CompilerParams(
            dimension_semantics=("arbitrary",)),
    )(cstart, cnum, A, B, batf, xs, w1, b1, w2, b2, offs)


def _node_body(h_ref, ag_ref, w2_ref, b2_ref, lw_ref, lb_ref, l1_ref,
               o_ref, xs_ref):
    x = jax.lax.dot(ag_ref[...], w2_ref[...], precision=HI) + b2_ref[...]
    x = _ssp(x)
    x = jax.lax.dot(x, lw_ref[...], precision=HI) + lb_ref[...]
    hn = h_ref[...] + x
    o_ref[...] = hn
    xs_ref[...] = jax.lax.dot(hn, l1_ref[...], precision=HI)


def _node(h, aggr, w2, b2, lw, lb, l1next):
    return pl.pallas_call(
        _node_body,
        grid=(N // RB,),
        in_specs=[
            pl.BlockSpec((RB, HIDDEN), lambda i: (i, 0)),
            pl.BlockSpec((RB, NF), lambda i: (i, 0)),
            pl.BlockSpec((NF, HIDDEN), lambda i: (0, 0)),
            pl.BlockSpec((1, HIDDEN), lambda i: (0, 0)),
            pl.BlockSpec((HIDDEN, HIDDEN), lambda i: (0, 0)),
            pl.BlockSpec((1, HIDDEN), lambda i: (0, 0)),
            pl.BlockSpec((HIDDEN, NF), lambda i: (0, 0)),
        ],
        out_specs=[
            pl.BlockSpec((RB, HIDDEN), lambda i: (i, 0)),
            pl.BlockSpec((RB, NF), lambda i: (i, 0)),
        ],
        out_shape=[
            jax.ShapeDtypeStruct((N, HIDDEN), jnp.float32),
            jax.ShapeDtypeStruct((N, NF), jnp.float32),
        ],
    )(h, aggr, w2, b2, lw, lb, l1next)


def _pool_body(bat_ref, h_ref, sum_ref, cnt_ref):
    i = pl.program_id(0)

    @pl.when(i == 0)
    def _():
        sum_ref[...] = jnp.zeros_like(sum_ref)
        cnt_ref[...] = jnp.zeros_like(cnt_ref)

    brow = bat_ref[...].reshape(1, RB)
    mol = jax.lax.broadcasted_iota(jnp.int32, (NMOL, RB), 0).astype(jnp.float32)
    mt = (mol == brow).astype(jnp.float32)
    sum_ref[...] += jax.lax.dot(mt, h_ref[...], precision=HI)
    cnt_ref[...] += jnp.sum(mt, axis=1, keepdims=True)


def _pool(batf, h):
    return pl.pallas_call(
        _pool_body,
        grid=(N // RB,),
        in_specs=[
            pl.BlockSpec((RB, 1), lambda i: (i, 0)),
            pl.BlockSpec((RB, HIDDEN), lambda i: (i, 0)),
        ],
        out_specs=[
            pl.BlockSpec((NMOL, HIDDEN), lambda i: (0, 0)),
            pl.BlockSpec((NMOL, 1), lambda i: (0, 0)),
        ],
        out_shape=[
            jax.ShapeDtypeStruct((NMOL, HIDDEN), jnp.float32),
            jax.ShapeDtypeStruct((NMOL, 1), jnp.float32),
        ],
    )(batf, h)


def _head_body(s_ref, c_ref, w1_ref, b1_ref, w2_ref, b2_ref, o_ref):
    g = s_ref[...] / jnp.maximum(c_ref[...], 1.0)
    z1 = jnp.maximum(jax.lax.dot(g, w1_ref[...], precision=HI) + b1_ref[...], 0.0)
    o_ref[...] = jax.lax.dot(z1, w2_ref[...], precision=HI) + b2_ref[...]


def _head(sums, cnts, w1, b1, w2, b2):
    return pl.pallas_call(
        _head_body,
        in_specs=[
            pl.BlockSpec((NMOL, HIDDEN), lambda: (0, 0)),
            pl.BlockSpec((NMOL, 1), lambda: (0, 0)),
            pl.BlockSpec((HIDDEN, HIDDEN), lambda: (0, 0)),
            pl.BlockSpec((1, HIDDEN), lambda: (0, 0)),
            pl.BlockSpec((HIDDEN, NT), lambda: (0, 0)),
            pl.BlockSpec((1, NT), lambda: (0, 0)),
        ],
        out_specs=pl.BlockSpec((NMOL, NT), lambda: (0, 0)),
        out_shape=jax.ShapeDtypeStruct((NMOL, NT), jnp.float32),
    )(sums, cnts, w1, b1, w2, b2)


def kernel(z, pos, batch, emb, mlp_w1, mlp_b1, mlp_w2, mlp_b2, lin1_w,
           lin2_w, lin2_b, lin_w, lin_b, cls_w1, cls_b1, cls_w2, cls_b2):
    pos = pos.astype(jnp.float32)
    x2 = jnp.sum(pos * pos, axis=1, keepdims=True)
    one = jnp.ones((N, 1), jnp.float32)
    zero3 = jnp.zeros((N, 3), jnp.float32)
    A = jnp.concatenate([-2.0 * pos, x2, one, zero3], axis=1)
    B = jnp.concatenate([pos, one, x2, zero3], axis=1)
    batf = batch.astype(jnp.float32).reshape(N, 1)

    r0s = jnp.arange(NRB, dtype=jnp.int32) * TR
    firstmol = batch[r0s]
    lastmol = batch[r0s + TR - 1]
    jmin = jnp.searchsorted(batch, firstmol, side="left").astype(jnp.int32)
    jmax = jnp.searchsorted(batch, lastmol, side="right").astype(jnp.int32)
    cstart = jmin // TC
    cnum = (jmax - 1) // TC - cstart + 1

    offs_pad = np.full((1, NGP), 1e6, np.float32) * np.float32(_K)
    offs_pad[0, :NG] = _OFFS_NP * np.float32(_K)
    offs = jnp.asarray(offs_pad)

    h = _embed_sc(z.astype(jnp.int32), emb)
    xs = _xs(h, lin1_w[0])
    for i in range(NI):
        w1p = (jnp.zeros((NGP, NF), jnp.float32).at[:NG].set(mlp_w1[i])
               .astype(jnp.bfloat16))
        w2b = mlp_w2[i].astype(jnp.bfloat16)
        b2f = (mlp_b2[i] - LOG2 * jnp.sum(w2b.astype(jnp.float32), axis=0))
        aggr = _pair(cstart, cnum, A, B, batf, xs,
                     w1p, mlp_b1[i].reshape(1, NF),
                     w2b, b2f.reshape(1, NF), offs)
        l1next = lin1_w[(i + 1) % NI]
        h, xs = _node(h, aggr, lin2_w[i], lin2_b[i].reshape(1, HIDDEN),
                      lin_w[i], lin_b[i].reshape(1, HIDDEN), l1next)

    sums, cnts = _pool(batf, h)
    return _head(sums, cnts, cls_w1, cls_b1.reshape(1, HIDDEN),
                 cls_w2, cls_b2.reshape(1, NT))


# speedup vs baseline: 1.5777x; 1.1160x over previous
import math

import jax
import jax.numpy as jnp
import numpy as np
from jax.experimental import pallas as pl
from jax.experimental.pallas import tpu as pltpu
from jax.experimental.pallas import tpu_sc as plsc

N = 8192
NMOL = 256
HIDDEN = 128
NF = 128
NI = 6
NG = 50
NGP = 128
CUTOFF = 10.0
NT = 12
TR = 32
TC = 32
TRS = 8
NRB = N // TR
RB = 128
NZ = 100
LOG2 = math.log(2.0)
HI = jax.lax.Precision.HIGHEST
_OFFS_NP = np.linspace(0.0, CUTOFF, NG).astype(np.float32)
_DELTA = _OFFS_NP[1] - _OFFS_NP[0]
_COEFF = float(np.float32(-0.5) / (_DELTA * _DELTA))


def _ssp(x):
    return jnp.maximum(x, 0.0) + jnp.log1p(jnp.exp(-jnp.abs(x))) - LOG2


_GW = 128


def _embed_sc(z, emb):
    mesh = plsc.VectorSubcoreMesh(core_axis_name="c", subcore_axis_name="s")

    @pl.kernel(out_type=jax.ShapeDtypeStruct((N, HIDDEN), jnp.float32),
               mesh=mesh)
    def gather_kernel(emb_hbm, zi_hbm, o_hbm):
        def body(i_vmem, o_vmem):
            pltpu.sync_copy(emb_hbm.at[i_vmem.at[0]], o_vmem)

        pltpu.emit_pipeline(
            body,
            grid=(N // _GW,),
            in_specs=[pl.BlockSpec((1, _GW), index_map=lambda i: (0, i))],
            out_specs=[pl.BlockSpec((_GW, HIDDEN), index_map=lambda i: (i, 0))],
            core_axis_name="s",
            dimension_semantics=(pltpu.PARALLEL,),
        )(zi_hbm, o_hbm)

    return gather_kernel(emb.astype(jnp.float32), z.reshape(1, N))


def _mm_body(x_ref, w_ref, o_ref):
    o_ref[...] = jax.lax.dot(x_ref[...], w_ref[...], precision=HI)


def _xs(h, w):
    return pl.pallas_call(
        _mm_body,
        grid=(N // RB,),
        in_specs=[
            pl.BlockSpec((RB, HIDDEN), lambda i: (i, 0)),
            pl.BlockSpec((HIDDEN, NF), lambda i: (0, 0)),
        ],
        out_specs=pl.BlockSpec((RB, NF), lambda i: (i, 0)),
        out_shape=jax.ShapeDtypeStruct((N, NF), jnp.float32),
    )(h, w)


_K = float(np.sqrt(np.float64(-_COEFF)))


def _sspl(x):
    return jnp.maximum(x, 0.0) + jnp.log1p(jnp.exp(-jnp.abs(x)))


def _pair_body(cs_ref, cn_ref, a_ref, b_ref, bat_ref, xs_ref,
               w1_ref, b1_ref, w2_ref, b2_ref, offs_ref, o_ref):
    i = pl.program_id(0)
    offs = offs_ref[...].reshape(1, 1, NGP)
    w1 = w1_ref[...]
    b1 = b1_ref[...]
    w2 = w2_ref[...]
    b2 = b2_ref[...]

    @pl.when(i == 0)
    def _():
        o_ref[...] = jnp.zeros_like(o_ref)

    for s in range(TRS):
        sb = i * TRS + s
        r0 = sb * TR
        a_blk = a_ref[pl.ds(s * TR, TR), :]
        bcol = bat_ref[pl.ds(r0, TR), :]
        row_ids = r0 + jax.lax.broadcasted_iota(jnp.int32, (TR, TC), 0)
        xsr = xs_ref[pl.ds(r0, TR), :]

        def body(t, acc):
            c0 = t * TC
            b_j = b_ref[pl.ds(c0, TC), :]
            d2 = jax.lax.dot_general(
                a_blk, b_j, (((1,), (1,)), ((), ())), precision=HI)
            brow = bat_ref[pl.ds(c0, TC), :].reshape(1, TC)
            col_ids = c0 + jax.lax.broadcasted_iota(jnp.int32, (TR, TC), 1)
            mask = (bcol == brow) & (d2 <= CUTOFF * CUTOFF) & (row_ids != col_ids)
            dm = jnp.where(mask, d2, 1e9)
            w = jnp.sqrt(jnp.maximum(dm, 0.0))
            cc = jnp.where(dm < 1e8,
                           0.5 * (jnp.cos(w * (math.pi / CUTOFF)) + 1.0), 0.0)
            ws3 = (w * _K).reshape(TR, TC, 1)
            ea = jnp.exp(-(ws3 - offs) ** 2)
            ea = ea.astype(jnp.bfloat16).reshape(TR * TC, NGP)
            a1 = (jax.lax.dot(ea, w1, preferred_element_type=jnp.float32)
                  + b1).astype(jnp.bfloat16)
            wf = jax.lax.dot(_sspl(a1), w2,
                             preferred_element_type=jnp.float32) + b2
            wf3 = wf.reshape(TR, TC, NF) * cc.reshape(TR, TC, 1)
            xsj = xs_ref[pl.ds(c0, TC), :]
            msg = wf3 * xsj[None, :, :]

            @pl.when(t > sb)
            def _():
                rmsg = wf3 * xsr[:, None, :]
                o_ref[pl.ds(c0, TC), :] += jnp.sum(rmsg, axis=0)

            return acc + jnp.sum(msg, axis=1)

        t0 = jnp.maximum(cs_ref[sb], sb)
        acc = jax.lax.fori_loop(t0, cs_ref[sb] + cn_ref[sb], body,
                                jnp.zeros((TR, NF), jnp.float32))
        o_ref[pl.ds(r0, TR), :] += acc


def _pair(cstart, cnum, A, B, batf, xs, w1, b1, w2, b2, offs):
    grid_spec = pltpu.PrefetchScalarGridSpec(
        num_scalar_prefetch=2,
        grid=(NRB // TRS,),
        in_specs=[
            pl.BlockSpec((TRS * TR, 8), lambda i, cs, cn: (i, 0)),
            pl.BlockSpec((N, 8), lambda i, cs, cn: (0, 0)),
            pl.BlockSpec((N, 1), lambda i, cs, cn: (0, 0)),
            pl.BlockSpec((N, NF), lambda i, cs, cn: (0, 0)),
            pl.BlockSpec((NGP, NF), lambda i, cs, cn: (0, 0)),
            pl.BlockSpec((1, NF), lambda i, cs, cn: (0, 0)),
            pl.BlockSpec((NF, NF), lambda i, cs, cn: (0, 0)),
            pl.BlockSpec((1, NF), lambda i, cs, cn: (0, 0)),
            pl.BlockSpec((1, NGP), lambda i, cs, cn: (0, 0)),
        ],
        out_specs=pl.BlockSpec((N, NF), lambda i, cs, cn: (0, 0)),
    )
    return pl.pallas_call(
        _pair_body,
        grid_spec=grid_spec,
        out_shape=jax.ShapeDtypeStruct((N, NF), jnp.float32),
        compiler_params=pltpu.CompilerParams(
            dimension_semantics=("arbitrary",)),
    )(cstart, cnum, A, B, batf, xs, w1, b1, w2, b2, offs)


def _node_body(h_ref, ag_ref, w2_ref, b2_ref, lw_ref, lb_ref, l1_ref,
               o_ref, xs_ref):
    x = jax.lax.dot(ag_ref[...], w2_ref[...], precision=HI) + b2_ref[...]
    x = _ssp(x)
    x = jax.lax.dot(x, lw_ref[...], precision=HI) + lb_ref[...]
    hn = h_ref[...] + x
    o_ref[...] = hn
    xs_ref[...] = jax.lax.dot(hn, l1_ref[...], precision=HI)


def _node(h, aggr, w2, b2, lw, lb, l1next):
    return pl.pallas_call(
        _node_body,
        grid=(N // RB,),
        in_specs=[
            pl.BlockSpec((RB, HIDDEN), lambda i: (i, 0)),
            pl.BlockSpec((RB, NF), lambda i: (i, 0)),
            pl.BlockSpec((NF, HIDDEN), lambda i: (0, 0)),
            pl.BlockSpec((1, HIDDEN), lambda i: (0, 0)),
            pl.BlockSpec((HIDDEN, HIDDEN), lambda i: (0, 0)),
            pl.BlockSpec((1, HIDDEN), lambda i: (0, 0)),
            pl.BlockSpec((HIDDEN, NF), lambda i: (0, 0)),
        ],
        out_specs=[
            pl.BlockSpec((RB, HIDDEN), lambda i: (i, 0)),
            pl.BlockSpec((RB, NF), lambda i: (i, 0)),
        ],
        out_shape=[
            jax.ShapeDtypeStruct((N, HIDDEN), jnp.float32),
            jax.ShapeDtypeStruct((N, NF), jnp.float32),
        ],
    )(h, aggr, w2, b2, lw, lb, l1next)


def _pool_body(bat_ref, h_ref, sum_ref, cnt_ref):
    i = pl.program_id(0)

    @pl.when(i == 0)
    def _():
        sum_ref[...] = jnp.zeros_like(sum_ref)
        cnt_ref[...] = jnp.zeros_like(cnt_ref)

    brow = bat_ref[...].reshape(1, RB)
    mol = jax.lax.broadcasted_iota(jnp.int32, (NMOL, RB), 0).astype(jnp.float32)
    mt = (mol == brow).astype(jnp.float32)
    sum_ref[...] += jax.lax.dot(mt, h_ref[...], precision=HI)
    cnt_ref[...] += jnp.sum(mt, axis=1, keepdims=True)


def _pool(batf, h):
    return pl.pallas_call(
        _pool_body,
        grid=(N // RB,),
        in_specs=[
            pl.BlockSpec((RB, 1), lambda i: (i, 0)),
            pl.BlockSpec((RB, HIDDEN), lambda i: (i, 0)),
        ],
        out_specs=[
            pl.BlockSpec((NMOL, HIDDEN), lambda i: (0, 0)),
            pl.BlockSpec((NMOL, 1), lambda i: (0, 0)),
        ],
        out_shape=[
            jax.ShapeDtypeStruct((NMOL, HIDDEN), jnp.float32),
            jax.ShapeDtypeStruct((NMOL, 1), jnp.float32),
        ],
    )(batf, h)


def _head_body(s_ref, c_ref, w1_ref, b1_ref, w2_ref, b2_ref, o_ref):
    g = s_ref[...] / jnp.maximum(c_ref[...], 1.0)
    z1 = jnp.maximum(jax.lax.dot(g, w1_ref[...], precision=HI) + b1_ref[...], 0.0)
    o_ref[...] = jax.lax.dot(z1, w2_ref[...], precision=HI) + b2_ref[...]


def _head(sums, cnts, w1, b1, w2, b2):
    return pl.pallas_call(
        _head_body,
        in_specs=[
            pl.BlockSpec((NMOL, HIDDEN), lambda: (0, 0)),
            pl.BlockSpec((NMOL, 1), lambda: (0, 0)),
            pl.BlockSpec((HIDDEN, HIDDEN), lambda: (0, 0)),
            pl.BlockSpec((1, HIDDEN), lambda: (0, 0)),
            pl.BlockSpec((HIDDEN, NT), lambda: (0, 0)),
            pl.BlockSpec((1, NT), lambda: (0, 0)),
        ],
        out_specs=pl.BlockSpec((NMOL, NT), lambda: (0, 0)),
        out_shape=jax.ShapeDtypeStruct((NMOL, NT), jnp.float32),
    )(sums, cnts, w1, b1, w2, b2)


def kernel(z, pos, batch, emb, mlp_w1, mlp_b1, mlp_w2, mlp_b2, lin1_w,
           lin2_w, lin2_b, lin_w, lin_b, cls_w1, cls_b1, cls_w2, cls_b2):
    pos = pos.astype(jnp.float32)
    x2 = jnp.sum(pos * pos, axis=1, keepdims=True)
    one = jnp.ones((N, 1), jnp.float32)
    zero3 = jnp.zeros((N, 3), jnp.float32)
    A = jnp.concatenate([-2.0 * pos, x2, one, zero3], axis=1)
    B = jnp.concatenate([pos, one, x2, zero3], axis=1)
    batf = batch.astype(jnp.float32).reshape(N, 1)

    r0s = jnp.arange(NRB, dtype=jnp.int32) * TR
    firstmol = batch[r0s]
    lastmol = batch[r0s + TR - 1]
    jmin = jnp.searchsorted(batch, firstmol, side="left").astype(jnp.int32)
    jmax = jnp.searchsorted(batch, lastmol, side="right").astype(jnp.int32)
    cstart = jmin // TC
    cnum = (jmax - 1) // TC - cstart + 1

    offs_pad = np.full((1, NGP), 1e6, np.float32) * np.float32(_K)
    offs_pad[0, :NG] = _OFFS_NP * np.float32(_K)
    offs = jnp.asarray(offs_pad)

    h = _embed_sc(z.astype(jnp.int32), emb)
    xs = _xs(h, lin1_w[0])
    for i in range(NI):
        w1p = (jnp.zeros((NGP, NF), jnp.float32).at[:NG].set(mlp_w1[i])
               .astype(jnp.bfloat16))
        w2b = mlp_w2[i].astype(jnp.bfloat16)
        b2f = (mlp_b2[i] - LOG2 * jnp.sum(w2b.astype(jnp.float32), axis=0))
        aggr = _pair(cstart, cnum, A, B, batf, xs,
                     w1p, mlp_b1[i].reshape(1, NF),
                     w2b, b2f.reshape(1, NF), offs)
        l1next = lin1_w[(i + 1) % NI]
        h, xs = _node(h, aggr, lin2_w[i], lin2_b[i].reshape(1, HIDDEN),
                      lin_w[i], lin_b[i].reshape(1, HIDDEN), l1next)

    sums, cnts = _pool(batf, h)
    return _head(sums, cnts, cls_w1, cls_b1.reshape(1, HIDDEN),
                 cls_w2, cls_b2.reshape(1, NT))
